# baseline (device time: 34395 ns/iter reference)
import jax
import jax.numpy as jnp
from jax import lax
from jax.experimental import pallas as pl
from jax.experimental.pallas import tpu as pltpu

N_DEV = 4
SEND_ORDER = (2, 1, 3)
ARRIVAL_ORDER = (1, 3, 2)


def _layer(x, win, wout, *, collective_id, first=False, last=False):
    b, d_shard = x.shape
    _, hdim = win.shape
    q = hdim // N_DEV

    def body(x_in, win_in, wout_hbm, out_ref, x_v, win_v, out_v, wout_v,
             pbuf, rs_buf, h_buf, ld_sems, wout_sem, rs_send, rs_recv,
             ag_send, ag_recv, out_sem):
        my_pos = lax.axis_index("i")

        if first:
            x_cp = pltpu.make_async_copy(x_in, x_v, ld_sems.at[N_DEV])
            x_cp.start()
            win_cp = {}
            order = list(SEND_ORDER) + [None]
            for j, k in enumerate(order):
                pos = (my_pos + k) % N_DEV if k is not None else my_pos
                cp = pltpu.make_async_copy(
                    win_in.at[:, pl.ds(pos * q, q)],
                    win_v.at[:, pl.ds(pos * q, q)],
                    ld_sems.at[j],
                )
                cp.start()
                win_cp[k] = cp
        wout_cp = pltpu.make_async_copy(wout_hbm, wout_v, wout_sem)
        wout_cp.start()

        barrier_sem = pltpu.get_barrier_semaphore()
        for k in range(1, N_DEV):
            pl.semaphore_signal(
                barrier_sem, inc=1,
                device_id=((my_pos + k) % N_DEV,),
                device_id_type=pl.DeviceIdType.MESH,
            )
        pl.semaphore_wait(barrier_sem, N_DEV - 1)

        if first:
            x_cp.wait()
            xv = x_v[...]
            win = win_v
        else:
            xv = x_in[...]
            win = win_in

        rs = {}
        for j, k in enumerate(SEND_ORDER):
            peer = (my_pos + k) % N_DEV
            if first:
                win_cp[k].wait()
            pbuf[k - 1] = jnp.dot(
                xv, win[:, pl.ds(peer * q, q)],
                preferred_element_type=jnp.float32).astype(jnp.bfloat16)
            rdma = pltpu.make_async_remote_copy(
                src_ref=pbuf.at[k - 1],
                dst_ref=rs_buf.at[k - 1],
                send_sem=rs_send.at[k - 1],
                recv_sem=rs_recv.at[k - 1],
                device_id=(peer,),
                device_id_type=pl.DeviceIdType.MESH,
            )
            rdma.start()
            rs[k] = rdma
        if first:
            win_cp[None].wait()
        own_q = jnp.dot(xv, win[:, pl.ds(my_pos * q, q)],
                        preferred_element_type=jnp.float32)

        hq = own_q
        for k in ARRIVAL_ORDER:
            rs[k].wait()
            hq = hq + rs_buf[k - 1].astype(jnp.float32)
        hq = jnp.maximum(hq, 0.0)
        h_buf[:, pl.ds(my_pos * q, q)] = hq.astype(jnp.bfloat16)

        ag = {}
        for k in SEND_ORDER:
            peer = (my_pos + k) % N_DEV
            rdma = pltpu.make_async_remote_copy(
                src_ref=h_buf.at[:, pl.ds(my_pos * q, q)],
                dst_ref=h_buf.at[:, pl.ds(my_pos * q, q)],
                send_sem=ag_send.at[k - 1],
                recv_sem=ag_recv.at[k - 1],
                device_id=(peer,),
                device_id_type=pl.DeviceIdType.MESH,
            )
            rdma.start()
            ag[k] = rdma

        wout_cp.wait()
        acc = jnp.dot(hq, wout_v[pl.ds(my_pos * q, q), :],
                      preferred_element_type=jnp.float32)
        for k in ARRIVAL_ORDER:
            ag[k].wait()
            src = (my_pos - k) % N_DEV
            acc = acc + jnp.dot(
                h_buf[:, pl.ds(src * q, q)].astype(jnp.float32),
                wout_v[pl.ds(src * q, q), :],
                preferred_element_type=jnp.float32)
        if last:
            out_v[...] = acc
            out_cp = pltpu.make_async_copy(out_v, out_ref, out_sem)
            out_cp.start()
            out_cp.wait()
        else:
            out_ref[...] = acc

    if first:
        x = pltpu.with_memory_space_constraint(x, pltpu.MemorySpace.HBM)
        win = pltpu.with_memory_space_constraint(win, pltpu.MemorySpace.HBM)
        xw_space = pltpu.MemorySpace.HBM
    else:
        xw_space = pltpu.MemorySpace.VMEM
    return pl.pallas_call(
        body,
        out_shape=jax.ShapeDtypeStruct((b, d_shard), jnp.float32),
        in_specs=[
            pl.BlockSpec(memory_space=xw_space),
            pl.BlockSpec(memory_space=xw_space),
            pl.BlockSpec(memory_space=pltpu.MemorySpace.HBM),
        ],
        out_specs=pl.BlockSpec(
            memory_space=pltpu.MemorySpace.HBM if last
            else pltpu.MemorySpace.VMEM),
        scratch_shapes=[
            pltpu.VMEM((b, d_shard), jnp.float32),
            pltpu.VMEM((d_shard, hdim), jnp.float32),
            pltpu.VMEM((b, d_shard), jnp.float32),
            pltpu.VMEM((hdim, d_shard), jnp.float32),
            pltpu.VMEM((N_DEV - 1, b, q), jnp.bfloat16),
            pltpu.VMEM((N_DEV - 1, b, q), jnp.bfloat16),
            pltpu.VMEM((b, hdim), jnp.bfloat16),
            pltpu.SemaphoreType.DMA((N_DEV + 1,)),
            pltpu.SemaphoreType.DMA,
            pltpu.SemaphoreType.DMA((N_DEV - 1,)),
            pltpu.SemaphoreType.DMA((N_DEV - 1,)),
            pltpu.SemaphoreType.DMA((N_DEV - 1,)),
            pltpu.SemaphoreType.DMA((N_DEV - 1,)),
            pltpu.SemaphoreType.DMA,
        ],
        compiler_params=pltpu.CompilerParams(collective_id=collective_id),
    )(x, win, pltpu.with_memory_space_constraint(
        wout, pltpu.MemorySpace.HBM))


def kernel(x, Win0, Wout0, Win1, Wout1, Win2, Wout2):
    x = _layer(x, Win0, Wout0, collective_id=0, first=True)
    x = _layer(x, Win1, Wout1, collective_id=1)
    x = _layer(x, Win2, Wout2, collective_id=2, last=True)
    return x


# device time: 31257 ns/iter; 1.1004x vs baseline; 1.1004x over previous
import jax
import jax.numpy as jnp
from jax import lax
from jax.experimental import pallas as pl
from jax.experimental.pallas import tpu as pltpu

N_DEV = 4
N_LAYERS = 3
SEND_ORDER = (2, 1, 3)
ARRIVAL_ORDER = (1, 3, 2)
LOAD_ORDER = (2, 1, 3, None)


def kernel(x, Win0, Wout0, Win1, Wout1, Win2, Wout2):
    b, d_shard = x.shape
    _, hdim = Win0.shape
    q = hdim // N_DEV

    def body(x_in, w0, w1, w2, o0, o1, o2, out_ref, x_v, win_v, wout_v,
             pbuf, rs_buf, h_buf, win_sems, wout_sem, rs_send, rs_recv,
             ag_send, ag_recv):
        my_pos = lax.axis_index("i")
        wins = [w0, w1, w2]
        wouts = [o0, o1, o2]

        def start_win_loads(l):
            cps = {}
            for j, k in enumerate(LOAD_ORDER):
                pos = (my_pos + k) % N_DEV if k is not None else my_pos
                cp = pltpu.make_async_copy(
                    wins[l].at[:, pl.ds(pos * q, q)],
                    win_v.at[:, pl.ds(pos * q, q)],
                    win_sems.at[j],
                )
                cp.start()
                cps[k] = cp
            return cps

        x_cp = pltpu.make_async_copy(x_in, x_v, win_sems.at[N_DEV])
        x_cp.start()
        win_cp = start_win_loads(0)
        wout_cp = pltpu.make_async_copy(wouts[0], wout_v, wout_sem)
        wout_cp.start()

        barrier_sem = pltpu.get_barrier_semaphore()
        for k in range(1, N_DEV):
            pl.semaphore_signal(
                barrier_sem, inc=1,
                device_id=((my_pos + k) % N_DEV,),
                device_id_type=pl.DeviceIdType.MESH,
            )
        pl.semaphore_wait(barrier_sem, N_DEV - 1)

        x_cp.wait()
        xv = x_v[...]

        for l in range(N_LAYERS):
            rs = {}
            for k in SEND_ORDER:
                peer = (my_pos + k) % N_DEV
                win_cp[k].wait()
                pbuf[k - 1] = jnp.dot(
                    xv, win_v[:, pl.ds(peer * q, q)],
                    preferred_element_type=jnp.float32).astype(jnp.bfloat16)
                rdma = pltpu.make_async_remote_copy(
                    src_ref=pbuf.at[k - 1],
                    dst_ref=rs_buf.at[k - 1],
                    send_sem=rs_send.at[k - 1],
                    recv_sem=rs_recv.at[k - 1],
                    device_id=(peer,),
                    device_id_type=pl.DeviceIdType.MESH,
                )
                rdma.start()
                rs[k] = rdma
            win_cp[None].wait()
            own_q = jnp.dot(xv, win_v[:, pl.ds(my_pos * q, q)],
                            preferred_element_type=jnp.float32)
            if l + 1 < N_LAYERS:
                win_cp = start_win_loads(l + 1)

            hq = own_q
            for k in ARRIVAL_ORDER:
                rs[k].wait()
                hq = hq + rs_buf[k - 1].astype(jnp.float32)
            hq = jnp.maximum(hq, 0.0)
            h_buf[:, pl.ds(my_pos * q, q)] = hq.astype(jnp.bfloat16)

            ag = {}
            for k in SEND_ORDER:
                peer = (my_pos + k) % N_DEV
                rdma = pltpu.make_async_remote_copy(
                    src_ref=h_buf.at[:, pl.ds(my_pos * q, q)],
                    dst_ref=h_buf.at[:, pl.ds(my_pos * q, q)],
                    send_sem=ag_send.at[k - 1],
                    recv_sem=ag_recv.at[k - 1],
                    device_id=(peer,),
                    device_id_type=pl.DeviceIdType.MESH,
                )
                rdma.start()
                ag[k] = rdma

            wout_cp.wait()
            acc = jnp.dot(hq, wout_v[pl.ds(my_pos * q, q), :],
                          preferred_element_type=jnp.float32)
            for k in ARRIVAL_ORDER:
                ag[k].wait()
                src = (my_pos - k) % N_DEV
                acc = acc + jnp.dot(
                    h_buf[:, pl.ds(src * q, q)].astype(jnp.float32),
                    wout_v[pl.ds(src * q, q), :],
                    preferred_element_type=jnp.float32)
            if l + 1 < N_LAYERS:
                wout_cp = pltpu.make_async_copy(wouts[l + 1], wout_v,
                                                wout_sem)
                wout_cp.start()
            xv = acc

        out_ref[...] = xv

    hbm = pltpu.MemorySpace.HBM
    args = [pltpu.with_memory_space_constraint(a, hbm)
            for a in (x, Win0, Win1, Win2, Wout0, Wout1, Wout2)]
    return pl.pallas_call(
        body,
        out_shape=jax.ShapeDtypeStruct((b, d_shard), jnp.float32),
        in_specs=[pl.BlockSpec(memory_space=hbm)] * 7,
        out_specs=pl.BlockSpec(memory_space=pltpu.MemorySpace.VMEM),
        scratch_shapes=[
            pltpu.VMEM((b, d_shard), jnp.float32),
            pltpu.VMEM((d_shard, hdim), jnp.float32),
            pltpu.VMEM((hdim, d_shard), jnp.float32),
            pltpu.VMEM((N_DEV - 1, b, q), jnp.bfloat16),
            pltpu.VMEM((N_DEV - 1, b, q), jnp.bfloat16),
            pltpu.VMEM((b, hdim), jnp.bfloat16),
            pltpu.SemaphoreType.DMA((N_DEV + 1,)),
            pltpu.SemaphoreType.DMA,
            pltpu.SemaphoreType.DMA((N_DEV - 1,)),
            pltpu.SemaphoreType.DMA((N_DEV - 1,)),
            pltpu.SemaphoreType.DMA((N_DEV - 1,)),
            pltpu.SemaphoreType.DMA((N_DEV - 1,)),
        ],
        compiler_params=pltpu.CompilerParams(collective_id=0),
    )(*args)


# device time: 31152 ns/iter; 1.1041x vs baseline; 1.0034x over previous
import jax
import jax.numpy as jnp
from jax import lax
from jax.experimental import pallas as pl
from jax.experimental.pallas import tpu as pltpu

N_DEV = 4
N_LAYERS = 3
SEND_ORDER = (2, 1, 3)
RS_ARRIVAL_ORDER = (1, 2, 3)
AG_ARRIVAL_ORDER = (1, 3, 2)
LOAD_ORDER = (2, 1, 3, None)


def kernel(x, Win0, Wout0, Win1, Wout1, Win2, Wout2):
    b, d_shard = x.shape
    _, hdim = Win0.shape
    q = hdim // N_DEV

    def body(x_in, w0, w1, w2, o0, o1, o2, out_ref, x_v, win_v, wout_v,
             pbuf, rs_buf, h_buf, win_sems, wout_sem, rs_send, rs_recv,
             ag_send, ag_recv):
        my_pos = lax.axis_index("i")
        wins = [w0, w1, w2]
        wouts = [o0, o1, o2]

        def start_win_loads(l):
            cps = {}
            for j, k in enumerate(LOAD_ORDER):
                pos = (my_pos + k) % N_DEV if k is not None else my_pos
                cp = pltpu.make_async_copy(
                    wins[l].at[:, pl.ds(pos * q, q)],
                    win_v.at[:, pl.ds(pos * q, q)],
                    win_sems.at[j],
                )
                cp.start()
                cps[k] = cp
            return cps

        x_cp = pltpu.make_async_copy(x_in, x_v, win_sems.at[N_DEV])
        x_cp.start()
        win_cp = start_win_loads(0)
        wout_cp = pltpu.make_async_copy(wouts[0], wout_v, wout_sem)
        wout_cp.start()

        barrier_sem = pltpu.get_barrier_semaphore()
        for k in range(1, N_DEV):
            pl.semaphore_signal(
                barrier_sem, inc=1,
                device_id=((my_pos + k) % N_DEV,),
                device_id_type=pl.DeviceIdType.MESH,
            )
        pl.semaphore_wait(barrier_sem, N_DEV - 1)

        x_cp.wait()
        xv = x_v[...]

        for l in range(N_LAYERS):
            rs = {}
            for k in SEND_ORDER:
                peer = (my_pos + k) % N_DEV
                win_cp[k].wait()
                pbuf[k - 1] = jnp.dot(
                    xv, win_v[:, pl.ds(peer * q, q)],
                    preferred_element_type=jnp.float32).astype(jnp.bfloat16)
                rdma = pltpu.make_async_remote_copy(
                    src_ref=pbuf.at[k - 1],
                    dst_ref=rs_buf.at[k - 1],
                    send_sem=rs_send.at[k - 1],
                    recv_sem=rs_recv.at[k - 1],
                    device_id=(peer,),
                    device_id_type=pl.DeviceIdType.MESH,
                )
                rdma.start()
                rs[k] = rdma
            win_cp[None].wait()
            own_q = jnp.dot(xv, win_v[:, pl.ds(my_pos * q, q)],
                            preferred_element_type=jnp.float32)
            if l + 1 < N_LAYERS:
                win_cp = start_win_loads(l + 1)

            hq = own_q
            for k in RS_ARRIVAL_ORDER:
                rs[k].wait()
                hq = hq + rs_buf[k - 1].astype(jnp.float32)
            hq = jnp.maximum(hq, 0.0)
            h_buf[:, pl.ds(my_pos * q, q)] = hq.astype(jnp.bfloat16)

            ag = {}
            for k in SEND_ORDER:
                peer = (my_pos + k) % N_DEV
                rdma = pltpu.make_async_remote_copy(
                    src_ref=h_buf.at[:, pl.ds(my_pos * q, q)],
                    dst_ref=h_buf.at[:, pl.ds(my_pos * q, q)],
                    send_sem=ag_send.at[k - 1],
                    recv_sem=ag_recv.at[k - 1],
                    device_id=(peer,),
                    device_id_type=pl.DeviceIdType.MESH,
                )
                rdma.start()
                ag[k] = rdma

            wout_cp.wait()
            acc = jnp.dot(hq, wout_v[pl.ds(my_pos * q, q), :],
                          preferred_element_type=jnp.float32)
            for k in AG_ARRIVAL_ORDER:
                ag[k].wait()
                src = (my_pos - k) % N_DEV
                acc = acc + jnp.dot(
                    h_buf[:, pl.ds(src * q, q)].astype(jnp.float32),
                    wout_v[pl.ds(src * q, q), :],
                    preferred_element_type=jnp.float32)
            if l + 1 < N_LAYERS:
                wout_cp = pltpu.make_async_copy(wouts[l + 1], wout_v,
                                                wout_sem)
                wout_cp.start()
            xv = acc

        out_ref[...] = xv

    hbm = pltpu.MemorySpace.HBM
    args = [pltpu.with_memory_space_constraint(a, hbm)
            for a in (x, Win0, Win1, Win2, Wout0, Wout1, Wout2)]
    return pl.pallas_call(
        body,
        out_shape=jax.ShapeDtypeStruct((b, d_shard), jnp.float32),
        in_specs=[pl.BlockSpec(memory_space=hbm)] * 7,
        out_specs=pl.BlockSpec(memory_space=pltpu.MemorySpace.VMEM),
        scratch_shapes=[
            pltpu.VMEM((b, d_shard), jnp.float32),
            pltpu.VMEM((d_shard, hdim), jnp.float32),
            pltpu.VMEM((hdim, d_shard), jnp.float32),
            pltpu.VMEM((N_DEV - 1, b, q), jnp.bfloat16),
            pltpu.VMEM((N_DEV - 1, b, q), jnp.bfloat16),
            pltpu.VMEM((b, hdim), jnp.bfloat16),
            pltpu.SemaphoreType.DMA((N_DEV + 1,)),
            pltpu.SemaphoreType.DMA,
            pltpu.SemaphoreType.DMA((N_DEV - 1,)),
            pltpu.SemaphoreType.DMA((N_DEV - 1,)),
            pltpu.SemaphoreType.DMA((N_DEV - 1,)),
            pltpu.SemaphoreType.DMA((N_DEV - 1,)),
        ],
        compiler_params=pltpu.CompilerParams(collective_id=0),
    )(*args)


# device time: 28215 ns/iter; 1.2190x vs baseline; 1.1041x over previous
import jax
import jax.numpy as jnp
from jax import lax
from jax.experimental import pallas as pl
from jax.experimental.pallas import tpu as pltpu

N_DEV = 4
N_LAYERS = 3
SEND_ORDER = (2, 1, 3)
ARRIVAL_ORDER = (1, 3, 2)
LOAD_ORDER = (2, 1, 3, None)


def kernel(x, Win0, Wout0, Win1, Wout1, Win2, Wout2):
    b, d_shard = x.shape
    _, hdim = Win0.shape
    q = hdim // N_DEV
    hh = q // 2

    def body(x_in, w0, w1, w2, o0, o1, o2, out_ref, x_v, win_v, wout_v,
             pbuf, rs_buf, h_buf, win_sems, wout_sem, rs_send, rs_recv,
             ag_send, ag_recv):
        my_pos = lax.axis_index("i")
        wins = [w0, w1, w2]
        wouts = [o0, o1, o2]

        def start_win_loads(l):
            cps = {}
            for j, k in enumerate(LOAD_ORDER):
                pos = (my_pos + k) % N_DEV if k is not None else my_pos
                cp = pltpu.make_async_copy(
                    wins[l].at[:, pl.ds(pos * q, q)],
                    win_v.at[:, pl.ds(pos * q, q)],
                    win_sems.at[j],
                )
                cp.start()
                cps[k] = cp
            return cps

        x_cp = pltpu.make_async_copy(x_in, x_v, win_sems.at[N_DEV])
        x_cp.start()
        win_cp = start_win_loads(0)
        wout_cp = pltpu.make_async_copy(wouts[0], wout_v, wout_sem)
        wout_cp.start()

        barrier_sem = pltpu.get_barrier_semaphore()
        for k in range(1, N_DEV):
            pl.semaphore_signal(
                barrier_sem, inc=1,
                device_id=((my_pos + k) % N_DEV,),
                device_id_type=pl.DeviceIdType.MESH,
            )
        pl.semaphore_wait(barrier_sem, N_DEV - 1)

        x_cp.wait()
        xv = x_v[...]

        for l in range(N_LAYERS):
            rs = {}
            for h in (0, 1):
                for k in SEND_ORDER:
                    peer = (my_pos + k) % N_DEV
                    if h == 0:
                        win_cp[k].wait()
                    slot = 2 * (k - 1) + h
                    pbuf[slot] = jnp.dot(
                        xv, win_v[:, pl.ds(peer * q + h * hh, hh)],
                        preferred_element_type=jnp.float32,
                    ).astype(jnp.bfloat16)
                    rdma = pltpu.make_async_remote_copy(
                        src_ref=pbuf.at[slot],
                        dst_ref=rs_buf.at[slot],
                        send_sem=rs_send.at[slot],
                        recv_sem=rs_recv.at[slot],
                        device_id=(peer,),
                        device_id_type=pl.DeviceIdType.MESH,
                    )
                    rdma.start()
                    rs[k, h] = rdma
            win_cp[None].wait()
            own_q = jnp.dot(xv, win_v[:, pl.ds(my_pos * q, q)],
                            preferred_element_type=jnp.float32)
            if l + 1 < N_LAYERS:
                win_cp = start_win_loads(l + 1)

            ag = {}
            hqs = []
            for h in (0, 1):
                hq = own_q[:, h * hh:(h + 1) * hh]
                for k in ARRIVAL_ORDER:
                    rs[k, h].wait()
                    hq = hq + rs_buf[2 * (k - 1) + h].astype(jnp.float32)
                hq = jnp.maximum(hq, 0.0)
                hqs.append(hq)
                col = my_pos * q + h * hh
                h_buf[:, pl.ds(col, hh)] = hq.astype(jnp.bfloat16)
                for k in SEND_ORDER:
                    peer = (my_pos + k) % N_DEV
                    slot = 2 * (k - 1) + h
                    rdma = pltpu.make_async_remote_copy(
                        src_ref=h_buf.at[:, pl.ds(col, hh)],
                        dst_ref=h_buf.at[:, pl.ds(col, hh)],
                        send_sem=ag_send.at[slot],
                        recv_sem=ag_recv.at[slot],
                        device_id=(peer,),
                        device_id_type=pl.DeviceIdType.MESH,
                    )
                    rdma.start()
                    ag[k, h] = rdma

            wout_cp.wait()
            acc = jnp.dot(hqs[0], wout_v[pl.ds(my_pos * q, hh), :],
                          preferred_element_type=jnp.float32)
            acc = acc + jnp.dot(hqs[1],
                                wout_v[pl.ds(my_pos * q + hh, hh), :],
                                preferred_element_type=jnp.float32)
            for h in (0, 1):
                for k in ARRIVAL_ORDER:
                    ag[k, h].wait()
                    src = (my_pos - k) % N_DEV
                    col = src * q + h * hh
                    acc = acc + jnp.dot(
                        h_buf[:, pl.ds(col, hh)].astype(jnp.float32),
                        wout_v[pl.ds(col, hh), :],
                        preferred_element_type=jnp.float32)
            if l + 1 < N_LAYERS:
                wout_cp = pltpu.make_async_copy(wouts[l + 1], wout_v,
                                                wout_sem)
                wout_cp.start()
            xv = acc

        out_ref[...] = xv

    hbm = pltpu.MemorySpace.HBM
    args = [pltpu.with_memory_space_constraint(a, hbm)
            for a in (x, Win0, Win1, Win2, Wout0, Wout1, Wout2)]
    n_slots = 2 * (N_DEV - 1)
    return pl.pallas_call(
        body,
        out_shape=jax.ShapeDtypeStruct((b, d_shard), jnp.float32),
        in_specs=[pl.BlockSpec(memory_space=hbm)] * 7,
        out_specs=pl.BlockSpec(memory_space=pltpu.MemorySpace.VMEM),
        scratch_shapes=[
            pltpu.VMEM((b, d_shard), jnp.float32),
            pltpu.VMEM((d_shard, hdim), jnp.float32),
            pltpu.VMEM((hdim, d_shard), jnp.float32),
            pltpu.VMEM((n_slots, b, hh), jnp.bfloat16),
            pltpu.VMEM((n_slots, b, hh), jnp.bfloat16),
            pltpu.VMEM((b, hdim), jnp.bfloat16),
            pltpu.SemaphoreType.DMA((N_DEV + 1,)),
            pltpu.SemaphoreType.DMA,
            pltpu.SemaphoreType.DMA((n_slots,)),
            pltpu.SemaphoreType.DMA((n_slots,)),
            pltpu.SemaphoreType.DMA((n_slots,)),
            pltpu.SemaphoreType.DMA((n_slots,)),
        ],
        compiler_params=pltpu.CompilerParams(collective_id=0),
    )(*args)
